# Initial kernel scaffold; baseline (speedup 1.0000x reference)
import jax
import jax.numpy as jnp
from jax.experimental import pallas as pl


def kernel(q, k, v, q_lat, q_lon, kv_lat, kv_lon):
    raise NotImplementedError("write your pallas kernel here")


# speedup vs baseline: 1.6020x; 1.6020x over previous
import functools

import jax
import jax.numpy as jnp
import numpy as np
from jax.experimental import pallas as pl
from jax.experimental.pallas import tpu as pltpu

_EARTH_RADIUS = 6371.0
_SPAN = 1500.0
_COS_THR = float(np.cos(_SPAN / _EARTH_RADIUS))
_NEG = float(np.finfo(np.float32).min)


def _flash_body(qlat_ref, qlon_ref, klat_ref, klon_ref, q_ref, k_ref, v_ref,
                o_ref):
    qlat = qlat_ref[0, :]
    qlon = qlon_ref[0, :]
    klat = klat_ref[0, :]
    klon = klon_ref[0, :]

    q_sl = jnp.sin(qlat)
    q_cl = jnp.cos(qlat)
    k_sl = jnp.sin(klat)
    k_cl = jnp.cos(klat)
    q_a = q_cl * jnp.sin(qlon)
    q_b = q_cl * jnp.cos(qlon)
    k_a = k_cl * jnp.sin(klon)
    k_b = k_cl * jnp.cos(klon)

    g = (q_sl[:, None] * k_sl[None, :]
         + q_a[:, None] * k_a[None, :]
         + q_b[:, None] * k_b[None, :])

    q = q_ref[0, 0]
    k = k_ref[0, 0]
    v = v_ref[0, 0]
    scale = 1.0 / np.sqrt(q.shape[-1]).astype(np.float32)

    s = jax.lax.dot_general(q, k, (((1,), (1,)), ((), ())),
                            preferred_element_type=jnp.float32) * scale
    s = jnp.where(g >= _COS_THR, s, _NEG)
    m = jnp.max(s, axis=1, keepdims=True)
    p = jnp.exp(s - m)
    denom = jnp.sum(p, axis=1, keepdims=True)
    o = jax.lax.dot_general(p, v, (((1,), (0,)), ((), ())),
                            preferred_element_type=jnp.float32)
    o_ref[0, 0] = o / denom


def kernel(q, k, v, q_lat, q_lon, kv_lat, kv_lon):
    B, H, S, D = q.shape
    Bq = 256
    nq = S // Bq

    qlat2 = q_lat.reshape(1, S)
    qlon2 = q_lon.reshape(1, S)
    klat2 = kv_lat.reshape(1, S)
    klon2 = kv_lon.reshape(1, S)

    grid = (H, nq)
    out = pl.pallas_call(
        _flash_body,
        grid=grid,
        in_specs=[
            pl.BlockSpec((1, Bq), lambda h, qi: (0, qi)),
            pl.BlockSpec((1, Bq), lambda h, qi: (0, qi)),
            pl.BlockSpec((1, S), lambda h, qi: (0, 0)),
            pl.BlockSpec((1, S), lambda h, qi: (0, 0)),
            pl.BlockSpec((1, 1, Bq, D), lambda h, qi: (0, h, qi, 0)),
            pl.BlockSpec((1, 1, S, D), lambda h, qi: (0, h, 0, 0)),
            pl.BlockSpec((1, 1, S, D), lambda h, qi: (0, h, 0, 0)),
        ],
        out_specs=pl.BlockSpec((1, 1, Bq, D), lambda h, qi: (0, h, qi, 0)),
        out_shape=jax.ShapeDtypeStruct((B, H, S, D), jnp.float32),
    )(qlat2, qlon2, klat2, klon2, q, k, v)
    return out


# speedup vs baseline: 1.6716x; 1.0434x over previous
import functools

import jax
import jax.numpy as jnp
import numpy as np
from jax.experimental import pallas as pl
from jax.experimental.pallas import tpu as pltpu

_EARTH_RADIUS = 6371.0
_SPAN = 1500.0
_COS_THR = float(np.cos(_SPAN / _EARTH_RADIUS))
_NEG = float(np.finfo(np.float32).min)


def _flash_body(qlat_ref, qlon_ref, klat_ref, klon_ref, q_ref, k_ref, v_ref,
                o_ref):
    qlat = qlat_ref[0, :]
    qlon = qlon_ref[0, :]
    klat = klat_ref[0, :]
    klon = klon_ref[0, :]

    q_sl = jnp.sin(qlat)
    q_cl = jnp.cos(qlat)
    k_sl = jnp.sin(klat)
    k_cl = jnp.cos(klat)
    q_a = q_cl * jnp.sin(qlon)
    q_b = q_cl * jnp.cos(qlon)
    k_a = k_cl * jnp.sin(klon)
    k_b = k_cl * jnp.cos(klon)

    g = (q_sl[:, None] * k_sl[None, :]
         + q_a[:, None] * k_a[None, :]
         + q_b[:, None] * k_b[None, :])

    q = q_ref[0, 0]
    k = k_ref[0, 0]
    v = v_ref[0, 0]
    scale = 1.0 / np.sqrt(q.shape[-1]).astype(np.float32)

    s = jax.lax.dot_general(q.astype(jnp.bfloat16), k.astype(jnp.bfloat16),
                            (((1,), (1,)), ((), ())),
                            preferred_element_type=jnp.float32) * scale
    s = jnp.where(g >= _COS_THR, s, _NEG)
    m = jnp.max(s, axis=1, keepdims=True)
    p = jnp.exp(s - m)
    denom = jnp.sum(p, axis=1, keepdims=True)
    o = jax.lax.dot_general(p.astype(jnp.bfloat16), v.astype(jnp.bfloat16),
                            (((1,), (0,)), ((), ())),
                            preferred_element_type=jnp.float32)
    o_ref[0, 0] = o / denom


def kernel(q, k, v, q_lat, q_lon, kv_lat, kv_lon):
    B, H, S, D = q.shape
    Bq = 256
    nq = S // Bq

    qlat2 = q_lat.reshape(1, S)
    qlon2 = q_lon.reshape(1, S)
    klat2 = kv_lat.reshape(1, S)
    klon2 = kv_lon.reshape(1, S)

    grid = (H, nq)
    out = pl.pallas_call(
        _flash_body,
        grid=grid,
        in_specs=[
            pl.BlockSpec((1, Bq), lambda h, qi: (0, qi)),
            pl.BlockSpec((1, Bq), lambda h, qi: (0, qi)),
            pl.BlockSpec((1, S), lambda h, qi: (0, 0)),
            pl.BlockSpec((1, S), lambda h, qi: (0, 0)),
            pl.BlockSpec((1, 1, Bq, D), lambda h, qi: (0, h, qi, 0)),
            pl.BlockSpec((1, 1, S, D), lambda h, qi: (0, h, 0, 0)),
            pl.BlockSpec((1, 1, S, D), lambda h, qi: (0, h, 0, 0)),
        ],
        out_specs=pl.BlockSpec((1, 1, Bq, D), lambda h, qi: (0, h, qi, 0)),
        out_shape=jax.ShapeDtypeStruct((B, H, S, D), jnp.float32),
    )(qlat2, qlon2, klat2, klon2, q, k, v)
    return out
